# baseline (speedup 1.0000x reference)
import functools

import jax
import jax.numpy as jnp
from jax.experimental import pallas as pl
from jax.experimental.pallas import tpu as pltpu

_SRMIN = 0.0
_SRMAX = 8.0
_BN = 8


def _embed_kernel(x_ref, idx8_ref, red_ref, tab_ref, out_ref, tile_ref, *,
                  bn, m_count, n_coeff, w_dim, n_grid, delta):
    p_total = bn * m_count

    u_inner = 16

    def _gather_chunk(k, carry):
        base = pl.multiple_of(k * u_inner, 8)
        for s in range(u_inner):
            i = pl.multiple_of(idx8_ref[0, 0, base + s], 8) + (s % 8)
            tile_ref[pl.ds(base + s, 1), :] = tab_ref[pl.ds(i, 1), :]
        return carry

    jax.lax.fori_loop(0, p_total // u_inner, _gather_chunk, 0)

    xr = x_ref[0][:, 0:1] - _SRMIN
    idx_f = jnp.floor(xr * (1.0 / delta))
    x0 = xr - idx_f * delta

    e = tile_ref[:, 0:w_dim]
    for i in range(1, n_coeff):
        e = e * x0 + tile_ref[:, i * w_dim:(i + 1) * w_dim]

    for a in range(bn):
        out_ref[0, a] = jnp.dot(red_ref[0, a],
                                e[a * m_count:(a + 1) * m_count, :],
                                preferred_element_type=jnp.float32)


def kernel(x, poly_coeff, reducer):
    n_atoms, m_count, _ = x.shape
    n_grid, n_coeff, w_dim = poly_coeff.shape
    r_dim = reducer.shape[1]
    delta = (_SRMAX - _SRMIN) / n_grid
    bn = _BN
    n_steps = n_atoms // bn
    half = n_steps // 2
    p_total = bn * m_count

    xr = x[..., 0] - _SRMIN
    idx8 = (xr * (1.0 / delta)).astype(jnp.int32) * 8
    tab = jnp.broadcast_to(
        poly_coeff.reshape(n_grid, 1, n_coeff * w_dim),
        (n_grid, 8, n_coeff * w_dim)).reshape(8 * n_grid, n_coeff * w_dim)

    x3 = x.reshape(n_steps, p_total, 1)
    idx3 = idx8.reshape(n_steps, 1, p_total)
    red3 = reducer.reshape(n_steps, bn, r_dim, m_count)

    out = pl.pallas_call(
        functools.partial(_embed_kernel, bn=bn, m_count=m_count,
                          n_coeff=n_coeff, w_dim=w_dim, n_grid=n_grid,
                          delta=delta),
        grid=(2, half),
        in_specs=[
            pl.BlockSpec((1, p_total, 1), lambda c, j: (c * half + j, 0, 0)),
            pl.BlockSpec((1, 1, p_total), lambda c, j: (c * half + j, 0, 0),
                         memory_space=pltpu.SMEM),
            pl.BlockSpec((1, bn, r_dim, m_count),
                         lambda c, j: (c * half + j, 0, 0, 0)),
            pl.BlockSpec((8 * n_grid, n_coeff * w_dim), lambda c, j: (0, 0)),
        ],
        out_specs=pl.BlockSpec((1, bn, r_dim, w_dim),
                               lambda c, j: (c * half + j, 0, 0, 0)),
        out_shape=jax.ShapeDtypeStruct((n_steps, bn, r_dim, w_dim),
                                       jnp.float32),
        scratch_shapes=[
            pltpu.VMEM((p_total, n_coeff * w_dim), jnp.float32),
        ],
        compiler_params=pltpu.CompilerParams(
            dimension_semantics=("parallel", "arbitrary"),
        ),
    )(x3, idx3, red3, tab)
    return out.reshape(n_atoms, r_dim, w_dim)


# speedup vs baseline: 1.5050x; 1.5050x over previous
import functools

import jax
import jax.numpy as jnp
from jax.experimental import pallas as pl
from jax.experimental.pallas import tpu as pltpu

_SRMIN = 0.0
_SRMAX = 8.0
_BN = 8


def _embed_kernel(x_ref, idx8_ref, red_ref, tab_ref, out_ref, tile_ref, *,
                  bn, m_count, n_coeff, w_dim, n_grid, delta):
    p_total = bn * m_count

    for m in range(p_total):
        i = pl.multiple_of(idx8_ref[0, 0, m], 8) + (m % 8)
        tile_ref[pl.ds(m, 1), :] = tab_ref[pl.ds(i, 1), :]

    xr = x_ref[0][:, 0:1] - _SRMIN
    idx_f = jnp.floor(xr * (1.0 / delta))
    x0 = xr - idx_f * delta

    e = tile_ref[:, 0:w_dim]
    for i in range(1, n_coeff):
        e = e * x0 + tile_ref[:, i * w_dim:(i + 1) * w_dim]

    for a in range(bn):
        out_ref[0, a] = jnp.dot(red_ref[0, a],
                                e[a * m_count:(a + 1) * m_count, :],
                                preferred_element_type=jnp.float32)


def kernel(x, poly_coeff, reducer):
    n_atoms, m_count, _ = x.shape
    n_grid, n_coeff, w_dim = poly_coeff.shape
    r_dim = reducer.shape[1]
    delta = (_SRMAX - _SRMIN) / n_grid
    bn = _BN
    n_steps = n_atoms // bn
    half = n_steps // 2
    p_total = bn * m_count

    xr = x[..., 0] - _SRMIN
    idx8 = (xr * (1.0 / delta)).astype(jnp.int32) * 8
    tab = jnp.broadcast_to(
        poly_coeff.reshape(n_grid, 1, n_coeff * w_dim),
        (n_grid, 8, n_coeff * w_dim)).reshape(8 * n_grid, n_coeff * w_dim)

    x3 = x.reshape(n_steps, p_total, 1)
    idx3 = idx8.reshape(n_steps, 1, p_total)
    red3 = reducer.reshape(n_steps, bn, r_dim, m_count)

    out = pl.pallas_call(
        functools.partial(_embed_kernel, bn=bn, m_count=m_count,
                          n_coeff=n_coeff, w_dim=w_dim, n_grid=n_grid,
                          delta=delta),
        grid=(2, half),
        in_specs=[
            pl.BlockSpec((1, p_total, 1), lambda c, j: (c * half + j, 0, 0)),
            pl.BlockSpec((1, 1, p_total), lambda c, j: (c * half + j, 0, 0),
                         memory_space=pltpu.SMEM),
            pl.BlockSpec((1, bn, r_dim, m_count),
                         lambda c, j: (c * half + j, 0, 0, 0)),
            pl.BlockSpec((8 * n_grid, n_coeff * w_dim), lambda c, j: (0, 0)),
        ],
        out_specs=pl.BlockSpec((1, bn, r_dim, w_dim),
                               lambda c, j: (c * half + j, 0, 0, 0)),
        out_shape=jax.ShapeDtypeStruct((n_steps, bn, r_dim, w_dim),
                                       jnp.float32),
        scratch_shapes=[
            pltpu.VMEM((p_total, n_coeff * w_dim), jnp.float32),
        ],
        compiler_params=pltpu.CompilerParams(
            dimension_semantics=("parallel", "arbitrary"),
        ),
    )(x3, idx3, red3, tab)
    return out.reshape(n_atoms, r_dim, w_dim)


# speedup vs baseline: 1.6467x; 1.0941x over previous
import functools

import jax
import jax.numpy as jnp
from jax.experimental import pallas as pl
from jax.experimental.pallas import tpu as pltpu

_SRMIN = 0.0
_SRMAX = 8.0
_BN = 8


def _embed_kernel(x_ref, idx8_ref, red_ref, tab_ref, out_ref, tile_ref, *,
                  bn, m_count, n_coeff, w_dim, n_grid, delta):
    p_total = bn * m_count

    for m in range(p_total):
        i = idx8_ref[0, 0, m]
        tile_ref[pl.ds(m, 1), :] = tab_ref[pl.ds(i, 1), :]

    xr = x_ref[0][:, 0:1] - _SRMIN
    idx_f = jnp.floor(xr * (1.0 / delta))
    x0 = xr - idx_f * delta

    e = tile_ref[:, 0:w_dim]
    for i in range(1, n_coeff):
        e = e * x0 + tile_ref[:, i * w_dim:(i + 1) * w_dim]

    for a in range(bn):
        out_ref[0, a] = jnp.dot(red_ref[0, a],
                                e[a * m_count:(a + 1) * m_count, :],
                                preferred_element_type=jnp.float32)


def kernel(x, poly_coeff, reducer):
    n_atoms, m_count, _ = x.shape
    n_grid, n_coeff, w_dim = poly_coeff.shape
    r_dim = reducer.shape[1]
    delta = (_SRMAX - _SRMIN) / n_grid
    bn = _BN
    n_steps = n_atoms // bn
    half = n_steps // 2
    p_total = bn * m_count

    xr = x[..., 0] - _SRMIN
    idx8 = (xr * (1.0 / delta)).astype(jnp.int32)
    tab = poly_coeff.reshape(n_grid, n_coeff * w_dim)

    x3 = x.reshape(n_steps, p_total, 1)
    idx3 = idx8.reshape(n_steps, 1, p_total)
    red3 = reducer.reshape(n_steps, bn, r_dim, m_count)

    out = pl.pallas_call(
        functools.partial(_embed_kernel, bn=bn, m_count=m_count,
                          n_coeff=n_coeff, w_dim=w_dim, n_grid=n_grid,
                          delta=delta),
        grid=(2, half),
        in_specs=[
            pl.BlockSpec((1, p_total, 1), lambda c, j: (c * half + j, 0, 0)),
            pl.BlockSpec((1, 1, p_total), lambda c, j: (c * half + j, 0, 0),
                         memory_space=pltpu.SMEM),
            pl.BlockSpec((1, bn, r_dim, m_count),
                         lambda c, j: (c * half + j, 0, 0, 0)),
            pl.BlockSpec((n_grid, n_coeff * w_dim), lambda c, j: (0, 0)),
        ],
        out_specs=pl.BlockSpec((1, bn, r_dim, w_dim),
                               lambda c, j: (c * half + j, 0, 0, 0)),
        out_shape=jax.ShapeDtypeStruct((n_steps, bn, r_dim, w_dim),
                                       jnp.float32),
        scratch_shapes=[
            pltpu.VMEM((p_total, n_coeff * w_dim), jnp.float32),
        ],
        compiler_params=pltpu.CompilerParams(
            dimension_semantics=("arbitrary", "arbitrary"),
        ),
    )(x3, idx3, red3, tab)
    return out.reshape(n_atoms, r_dim, w_dim)


# speedup vs baseline: 1.7260x; 1.0482x over previous
import functools

import jax
import jax.numpy as jnp
from jax.experimental import pallas as pl
from jax.experimental.pallas import tpu as pltpu

_SRMIN = 0.0
_SRMAX = 8.0
_BN = 8


def _embed_kernel(x_ref, idx8_ref, red_ref, tab_ref, out_ref, *tile_refs,
                  bn, m_count, n_coeff, w_dim, n_grid, delta):
    xr = x_ref[0][:, 0:1] - _SRMIN
    idx_f = jnp.floor(xr * (1.0 / delta))
    x0 = xr - idx_f * delta

    for a in range(bn):
        tile_ref = tile_refs[a]
        for s in range(m_count):
            i = idx8_ref[0, 0, a * m_count + s]
            tile_ref[pl.ds(s, 1), :] = tab_ref[pl.ds(i, 1), :]

        x0a = x0[a * m_count:(a + 1) * m_count, :]
        e = tile_ref[:, 0:w_dim]
        for i in range(1, n_coeff):
            e = e * x0a + tile_ref[:, i * w_dim:(i + 1) * w_dim]

        out_ref[0, a] = jnp.dot(red_ref[0, a], e,
                                preferred_element_type=jnp.float32)


def kernel(x, poly_coeff, reducer):
    n_atoms, m_count, _ = x.shape
    n_grid, n_coeff, w_dim = poly_coeff.shape
    r_dim = reducer.shape[1]
    delta = (_SRMAX - _SRMIN) / n_grid
    bn = _BN
    n_steps = n_atoms // bn
    half = n_steps // 2
    p_total = bn * m_count

    xr = x[..., 0] - _SRMIN
    idx8 = (xr * (1.0 / delta)).astype(jnp.int32)
    tab = poly_coeff.reshape(n_grid, n_coeff * w_dim)

    x3 = x.reshape(n_steps, p_total, 1)
    idx3 = idx8.reshape(n_steps, 1, p_total)
    red3 = reducer.reshape(n_steps, bn, r_dim, m_count)

    out = pl.pallas_call(
        functools.partial(_embed_kernel, bn=bn, m_count=m_count,
                          n_coeff=n_coeff, w_dim=w_dim, n_grid=n_grid,
                          delta=delta),
        grid=(2, half),
        in_specs=[
            pl.BlockSpec((1, p_total, 1), lambda c, j: (c * half + j, 0, 0)),
            pl.BlockSpec((1, 1, p_total), lambda c, j: (c * half + j, 0, 0),
                         memory_space=pltpu.SMEM),
            pl.BlockSpec((1, bn, r_dim, m_count),
                         lambda c, j: (c * half + j, 0, 0, 0)),
            pl.BlockSpec((n_grid, n_coeff * w_dim), lambda c, j: (0, 0)),
        ],
        out_specs=pl.BlockSpec((1, bn, r_dim, w_dim),
                               lambda c, j: (c * half + j, 0, 0, 0)),
        out_shape=jax.ShapeDtypeStruct((n_steps, bn, r_dim, w_dim),
                                       jnp.float32),
        scratch_shapes=[
            pltpu.VMEM((m_count, n_coeff * w_dim), jnp.float32)
            for _ in range(bn)
        ],
        compiler_params=pltpu.CompilerParams(
            dimension_semantics=("arbitrary", "arbitrary"),
        ),
    )(x3, idx3, red3, tab)
    return out.reshape(n_atoms, r_dim, w_dim)


# speedup vs baseline: 1.8094x; 1.0483x over previous
import functools

import jax
import jax.numpy as jnp
from jax.experimental import pallas as pl
from jax.experimental.pallas import tpu as pltpu

_SRMIN = 0.0
_SRMAX = 8.0
_BN = 16


def _embed_kernel(x_ref, idx8_ref, red_ref, tab_ref, out_ref, *tile_refs,
                  bn, m_count, n_coeff, w_dim, n_grid, delta):
    xr = x_ref[0][:, 0:1] - _SRMIN
    idx_f = jnp.floor(xr * (1.0 / delta))
    x0 = xr - idx_f * delta

    for a in range(bn):
        tile_ref = tile_refs[a]
        for s in range(m_count):
            i = idx8_ref[0, 0, a * m_count + s]
            tile_ref[pl.ds(s, 1), :] = tab_ref[pl.ds(i, 1), :]

        x0a = x0[a * m_count:(a + 1) * m_count, :]
        e = tile_ref[:, 0:w_dim]
        for i in range(1, n_coeff):
            e = e * x0a + tile_ref[:, i * w_dim:(i + 1) * w_dim]

        out_ref[0, a] = jnp.dot(red_ref[0, a], e,
                                preferred_element_type=jnp.float32)


def kernel(x, poly_coeff, reducer):
    n_atoms, m_count, _ = x.shape
    n_grid, n_coeff, w_dim = poly_coeff.shape
    r_dim = reducer.shape[1]
    delta = (_SRMAX - _SRMIN) / n_grid
    bn = _BN
    n_steps = n_atoms // bn
    half = n_steps // 2
    p_total = bn * m_count

    xr = x[..., 0] - _SRMIN
    idx8 = (xr * (1.0 / delta)).astype(jnp.int32)
    tab = poly_coeff.reshape(n_grid, n_coeff * w_dim)

    x3 = x.reshape(n_steps, p_total, 1)
    idx3 = idx8.reshape(n_steps, 1, p_total)
    red3 = reducer.reshape(n_steps, bn, r_dim, m_count)

    out = pl.pallas_call(
        functools.partial(_embed_kernel, bn=bn, m_count=m_count,
                          n_coeff=n_coeff, w_dim=w_dim, n_grid=n_grid,
                          delta=delta),
        grid=(2, half),
        in_specs=[
            pl.BlockSpec((1, p_total, 1), lambda c, j: (c * half + j, 0, 0)),
            pl.BlockSpec((1, 1, p_total), lambda c, j: (c * half + j, 0, 0),
                         memory_space=pltpu.SMEM),
            pl.BlockSpec((1, bn, r_dim, m_count),
                         lambda c, j: (c * half + j, 0, 0, 0)),
            pl.BlockSpec((n_grid, n_coeff * w_dim), lambda c, j: (0, 0)),
        ],
        out_specs=pl.BlockSpec((1, bn, r_dim, w_dim),
                               lambda c, j: (c * half + j, 0, 0, 0)),
        out_shape=jax.ShapeDtypeStruct((n_steps, bn, r_dim, w_dim),
                                       jnp.float32),
        scratch_shapes=[
            pltpu.VMEM((m_count, n_coeff * w_dim), jnp.float32)
            for _ in range(bn)
        ],
        compiler_params=pltpu.CompilerParams(
            dimension_semantics=("arbitrary", "arbitrary"),
        ),
    )(x3, idx3, red3, tab)
    return out.reshape(n_atoms, r_dim, w_dim)


# speedup vs baseline: 1.8523x; 1.0237x over previous
import functools

import jax
import jax.numpy as jnp
from jax.experimental import pallas as pl
from jax.experimental.pallas import tpu as pltpu

_SRMIN = 0.0
_SRMAX = 8.0
_BN = 32


def _embed_kernel(x_ref, idx8_ref, red_ref, tab_ref, out_ref, *tile_refs,
                  bn, m_count, n_coeff, w_dim, n_grid, delta):
    xr = x_ref[0][:, 0:1] - _SRMIN
    idx_f = jnp.floor(xr * (1.0 / delta))
    x0 = xr - idx_f * delta

    for a in range(bn):
        tile_ref = tile_refs[a]
        for s in range(m_count):
            i = idx8_ref[0, 0, a * m_count + s]
            tile_ref[pl.ds(s, 1), :] = tab_ref[pl.ds(i, 1), :]

        x0a = x0[a * m_count:(a + 1) * m_count, :]
        e = tile_ref[:, 0:w_dim]
        for i in range(1, n_coeff):
            e = e * x0a + tile_ref[:, i * w_dim:(i + 1) * w_dim]

        out_ref[0, a] = jnp.dot(red_ref[0, a], e,
                                preferred_element_type=jnp.float32)


def kernel(x, poly_coeff, reducer):
    n_atoms, m_count, _ = x.shape
    n_grid, n_coeff, w_dim = poly_coeff.shape
    r_dim = reducer.shape[1]
    delta = (_SRMAX - _SRMIN) / n_grid
    bn = _BN
    n_steps = n_atoms // bn
    half = n_steps // 2
    p_total = bn * m_count

    xr = x[..., 0] - _SRMIN
    idx8 = (xr * (1.0 / delta)).astype(jnp.int32)
    tab = poly_coeff.reshape(n_grid, n_coeff * w_dim)

    x3 = x.reshape(n_steps, p_total, 1)
    idx3 = idx8.reshape(n_steps, 1, p_total)
    red3 = reducer.reshape(n_steps, bn, r_dim, m_count)

    out = pl.pallas_call(
        functools.partial(_embed_kernel, bn=bn, m_count=m_count,
                          n_coeff=n_coeff, w_dim=w_dim, n_grid=n_grid,
                          delta=delta),
        grid=(2, half),
        in_specs=[
            pl.BlockSpec((1, p_total, 1), lambda c, j: (c * half + j, 0, 0)),
            pl.BlockSpec((1, 1, p_total), lambda c, j: (c * half + j, 0, 0),
                         memory_space=pltpu.SMEM),
            pl.BlockSpec((1, bn, r_dim, m_count),
                         lambda c, j: (c * half + j, 0, 0, 0)),
            pl.BlockSpec((n_grid, n_coeff * w_dim), lambda c, j: (0, 0)),
        ],
        out_specs=pl.BlockSpec((1, bn, r_dim, w_dim),
                               lambda c, j: (c * half + j, 0, 0, 0)),
        out_shape=jax.ShapeDtypeStruct((n_steps, bn, r_dim, w_dim),
                                       jnp.float32),
        scratch_shapes=[
            pltpu.VMEM((m_count, n_coeff * w_dim), jnp.float32)
            for _ in range(bn)
        ],
        compiler_params=pltpu.CompilerParams(
            dimension_semantics=("arbitrary", "arbitrary"),
        ),
    )(x3, idx3, red3, tab)
    return out.reshape(n_atoms, r_dim, w_dim)


# speedup vs baseline: 1.8739x; 1.0117x over previous
import functools

import jax
import jax.numpy as jnp
from jax.experimental import pallas as pl
from jax.experimental.pallas import tpu as pltpu

_SRMIN = 0.0
_SRMAX = 8.0
_BN = 64


def _embed_kernel(x_ref, idx8_ref, red_ref, tab_ref, out_ref, *tile_refs,
                  bn, m_count, n_coeff, w_dim, n_grid, delta):
    xr = x_ref[0][:, 0:1] - _SRMIN
    idx_f = jnp.floor(xr * (1.0 / delta))
    x0 = xr - idx_f * delta

    for a in range(bn):
        tile_ref = tile_refs[a]
        for s in range(m_count):
            i = idx8_ref[0, 0, a * m_count + s]
            tile_ref[pl.ds(s, 1), :] = tab_ref[pl.ds(i, 1), :]

        x0a = x0[a * m_count:(a + 1) * m_count, :]
        e = tile_ref[:, 0:w_dim]
        for i in range(1, n_coeff):
            e = e * x0a + tile_ref[:, i * w_dim:(i + 1) * w_dim]

        out_ref[0, a] = jnp.dot(red_ref[0, a], e,
                                preferred_element_type=jnp.float32)


def kernel(x, poly_coeff, reducer):
    n_atoms, m_count, _ = x.shape
    n_grid, n_coeff, w_dim = poly_coeff.shape
    r_dim = reducer.shape[1]
    delta = (_SRMAX - _SRMIN) / n_grid
    bn = _BN
    n_steps = n_atoms // bn
    half = n_steps // 2
    p_total = bn * m_count

    xr = x[..., 0] - _SRMIN
    idx8 = (xr * (1.0 / delta)).astype(jnp.int32)
    tab = poly_coeff.reshape(n_grid, n_coeff * w_dim)

    x3 = x.reshape(n_steps, p_total, 1)
    idx3 = idx8.reshape(n_steps, 1, p_total)
    red3 = reducer.reshape(n_steps, bn, r_dim, m_count)

    out = pl.pallas_call(
        functools.partial(_embed_kernel, bn=bn, m_count=m_count,
                          n_coeff=n_coeff, w_dim=w_dim, n_grid=n_grid,
                          delta=delta),
        grid=(2, half),
        in_specs=[
            pl.BlockSpec((1, p_total, 1), lambda c, j: (c * half + j, 0, 0)),
            pl.BlockSpec((1, 1, p_total), lambda c, j: (c * half + j, 0, 0),
                         memory_space=pltpu.SMEM),
            pl.BlockSpec((1, bn, r_dim, m_count),
                         lambda c, j: (c * half + j, 0, 0, 0)),
            pl.BlockSpec((n_grid, n_coeff * w_dim), lambda c, j: (0, 0)),
        ],
        out_specs=pl.BlockSpec((1, bn, r_dim, w_dim),
                               lambda c, j: (c * half + j, 0, 0, 0)),
        out_shape=jax.ShapeDtypeStruct((n_steps, bn, r_dim, w_dim),
                                       jnp.float32),
        scratch_shapes=[
            pltpu.VMEM((m_count, n_coeff * w_dim), jnp.float32)
            for _ in range(bn)
        ],
        compiler_params=pltpu.CompilerParams(
            dimension_semantics=("arbitrary", "arbitrary"),
        ),
    )(x3, idx3, red3, tab)
    return out.reshape(n_atoms, r_dim, w_dim)
